# baseline (speedup 1.0000x reference)
import jax
import jax.numpy as jnp
from jax.experimental import pallas as pl


def _selfmix_kernel(x_ref, keep_ref, mix_ref, o_ref):
    xb = x_ref[...]
    y = jax.lax.dot_general(
        xb, mix_ref[...], (((1,), (1,)), ((), ())),
        preferred_element_type=jnp.float32,
    )
    o_ref[...] = xb * (keep_ref[...] + 0.5 * y)


def kernel(x, keep_coeff, mix_coeff):
    n, c = x.shape
    mix = mix_coeff.reshape(c, c)
    keep = keep_coeff.reshape(1, c)
    blk = 256
    grid = n // blk
    return pl.pallas_call(
        _selfmix_kernel,
        out_shape=jax.ShapeDtypeStruct((n, c), x.dtype),
        grid=(grid,),
        in_specs=[
            pl.BlockSpec((blk, c), lambda i: (i, 0)),
            pl.BlockSpec((1, c), lambda i: (0, 0)),
            pl.BlockSpec((c, c), lambda i: (0, 0)),
        ],
        out_specs=pl.BlockSpec((blk, c), lambda i: (i, 0)),
    )(x, keep, mix)


# speedup vs baseline: 2.3478x; 2.3478x over previous
import jax
import jax.numpy as jnp
from jax.experimental import pallas as pl


def _selfmix_kernel(x_ref, keep_ref, mix_ref, o_ref):
    xb = x_ref[...]
    y = jax.lax.dot_general(
        xb, mix_ref[...], (((1,), (1,)), ((), ())),
        preferred_element_type=jnp.float32,
    )
    o_ref[...] = xb * (keep_ref[...] + 0.5 * y)


def kernel(x, keep_coeff, mix_coeff):
    n, c = x.shape
    mix = mix_coeff.reshape(c, c)
    keep = keep_coeff.reshape(1, c)
    blk = 1024
    grid = n // blk
    return pl.pallas_call(
        _selfmix_kernel,
        out_shape=jax.ShapeDtypeStruct((n, c), x.dtype),
        grid=(grid,),
        in_specs=[
            pl.BlockSpec((blk, c), lambda i: (i, 0)),
            pl.BlockSpec((1, c), lambda i: (0, 0)),
            pl.BlockSpec((c, c), lambda i: (0, 0)),
        ],
        out_specs=pl.BlockSpec((blk, c), lambda i: (i, 0)),
    )(x, keep, mix)
